# Initial kernel scaffold; baseline (speedup 1.0000x reference)
import jax
import jax.numpy as jnp
from jax.experimental import pallas as pl


def kernel(x, shift_param):
    raise NotImplementedError("write your pallas kernel here")


# speedup vs baseline: 7.1886x; 7.1886x over previous
import functools
import jax
import jax.numpy as jnp
from jax import lax
from jax.experimental import pallas as pl
from jax.experimental.pallas import tpu as pltpu
from jax.experimental.pallas import tpu_sc as plsc

MAX_STEPS = 204.0
L = 16


def _make_sc_shift(R, T, C):
    info = plsc.get_sparse_core_info()
    NC, NS = info.num_cores, info.num_subcores
    NW = NC * NS
    assert R % NW == 0 and C % L == 0 and T % (8 * L) == 0
    rows_per = R // NW

    mesh = plsc.VectorSubcoreMesh(core_axis_name="c", subcore_axis_name="s")

    @functools.partial(
        pl.kernel,
        mesh=mesh,
        out_type=jax.ShapeDtypeStruct((R, T), jnp.float32),
        compiler_params=pltpu.CompilerParams(needs_layout_passes=False),
        scratch_types=[
            pltpu.VMEM((C,), jnp.float32),
            pltpu.VMEM((C,), jnp.int32),
            pltpu.VMEM((C,), jnp.float32),
            pltpu.VMEM((T,), jnp.float32),
            pltpu.VMEM((T,), jnp.float32),
        ],
    )
    def sc_shift(x_hbm, shift_hbm, out_hbm, shiftv, kbuf, abuf, inbuf, obuf):
        wid = lax.axis_index("s") * NC + lax.axis_index("c")
        pltpu.sync_copy(shift_hbm, shiftv)

        for i in range(C // L):
            p = shiftv[pl.ds(i * L, L)]
            e = jnp.exp(-2.0 * jnp.abs(p))
            s = jnp.sign(p) * ((1.0 - e) / (1.0 + e)) * MAX_STEPS
            tr = s.astype(jnp.int32)
            kf = jnp.where(tr.astype(jnp.float32) > s, tr - 1, tr)
            kbuf[pl.ds(i * L, L)] = kf
            abuf[pl.ds(i * L, L)] = s - kf.astype(jnp.float32)

        iota = lax.iota(jnp.int32, L)

        def row_body(j, _):
            row = wid * rows_per + j
            ch = jnp.full((L,), lax.rem(row, C), jnp.int32)
            av = plsc.load_gather(abuf, [ch])
            bv = 1.0 - av
            base = plsc.load_gather(kbuf, [ch]) + iota

            pltpu.sync_copy(x_hbm.at[row], inbuf)

            U = 8

            def chunk_body(ci, _):
                t0 = ci * (U * L)
                for u in range(U):
                    t = pl.multiple_of(t0 + u * L, L)
                    idx = base + t
                    i0 = jnp.minimum(jnp.maximum(idx, 0), T - 1)
                    i1 = jnp.minimum(jnp.maximum(idx + 1, 0), T - 1)
                    v0 = plsc.load_gather(inbuf, [i0])
                    v1 = plsc.load_gather(inbuf, [i1])
                    obuf[pl.ds(t, L)] = bv * v0 + av * v1
                return 0

            lax.fori_loop(0, T // (U * L), chunk_body, 0)
            pltpu.sync_copy(obuf, out_hbm.at[row])
            return 0

        lax.fori_loop(0, rows_per, row_body, 0)

    return sc_shift


def kernel(x, shift_param):
    B, C, T = x.shape
    xr = x.reshape(B * C, T)
    sp = shift_param.reshape(C).astype(jnp.float32)
    out = _make_sc_shift(B * C, T, C)(xr, sp)
    return out.reshape(B, C, T)


# speedup vs baseline: 9.9776x; 1.3880x over previous
import functools
import jax
import jax.numpy as jnp
from jax import lax
from jax.experimental import pallas as pl
from jax.experimental.pallas import tpu as pltpu
from jax.experimental.pallas import tpu_sc as plsc

MAX_STEPS = 204.0
L = 16
HEAD = 16
TAIL = 16
U = 8


def _make_sc_shift(R, T, C):
    info = plsc.get_sparse_core_info()
    NC, NS = info.num_cores, info.num_subcores
    NW = NC * NS
    assert R % (2 * NW) == 0 and C % L == 0
    nchunks = T // L
    n_int = nchunks - HEAD - TAIL
    assert T % L == 0 and n_int % U == 0 and HEAD * L >= MAX_STEPS + 1
    rows_per = R // NW
    pairs = rows_per // 2

    mesh = plsc.VectorSubcoreMesh(core_axis_name="c", subcore_axis_name="s")

    @functools.partial(
        pl.kernel,
        mesh=mesh,
        out_type=jax.ShapeDtypeStruct((R, T), jnp.float32),
        compiler_params=pltpu.CompilerParams(needs_layout_passes=False),
        scratch_types=[
            pltpu.VMEM((C,), jnp.float32),
            pltpu.VMEM((C,), jnp.int32),
            pltpu.VMEM((C,), jnp.float32),
            pltpu.VMEM((T,), jnp.float32),
            pltpu.VMEM((T,), jnp.float32),
            pltpu.VMEM((T,), jnp.float32),
            pltpu.VMEM((T,), jnp.float32),
            pltpu.SemaphoreType.DMA,
            pltpu.SemaphoreType.DMA,
            pltpu.SemaphoreType.DMA,
            pltpu.SemaphoreType.DMA,
        ],
    )
    def sc_shift(x_hbm, shift_hbm, out_hbm, shiftv, kbuf, abuf,
                 in0, in1, ob0, ob1, si0, si1, so0, so1):
        wid = lax.axis_index("s") * NC + lax.axis_index("c")
        pltpu.sync_copy(shift_hbm, shiftv)

        for i in range(C // L):
            p = shiftv[pl.ds(i * L, L)]
            e = jnp.exp(-2.0 * jnp.abs(p))
            s = jnp.sign(p) * ((1.0 - e) / (1.0 + e)) * MAX_STEPS
            tr = s.astype(jnp.int32)
            kf = jnp.where(tr.astype(jnp.float32) > s, tr - 1, tr)
            kbuf[pl.ds(i * L, L)] = kf
            abuf[pl.ds(i * L, L)] = s - kf.astype(jnp.float32)

        iota = lax.iota(jnp.int32, L)
        base_row = wid * rows_per

        def row_params(row):
            ch = jnp.full((L,), lax.rem(row, C), jnp.int32)
            av = plsc.load_gather(abuf, [ch])
            base = plsc.load_gather(kbuf, [ch]) + iota
            return av, 1.0 - av, base

        def clip_chunk(inb, ob, av, bv, base, t):
            idx = base + t
            i0 = jnp.minimum(jnp.maximum(idx, 0), T - 1)
            i1 = jnp.minimum(jnp.maximum(idx + 1, 0), T - 1)
            v0 = plsc.load_gather(inb, [i0])
            v1 = plsc.load_gather(inb, [i1])
            ob[pl.ds(t, L)] = bv * v0 + av * v1

        def compute_row(inb, ob, av, bv, base):
            for ci in range(HEAD):
                clip_chunk(inb, ob, av, bv, base, ci * L)

            def ibody(i, _):
                t0 = HEAD * L + i * (U * L)
                for u in range(U):
                    t = t0 + u * L
                    i0 = base + t
                    v0 = plsc.load_gather(inb, [i0])
                    v1 = plsc.load_gather(inb, [i0 + 1])
                    ob[pl.ds(pl.multiple_of(t, L), L)] = bv * v0 + av * v1
                return 0

            lax.fori_loop(0, n_int // U, ibody, 0)
            for ci in range(nchunks - TAIL, nchunks):
                clip_chunk(inb, ob, av, bv, base, ci * L)

        pltpu.make_async_copy(x_hbm.at[base_row], in0, si0).start()

        def pair_body(p, _):
            r0 = base_row + 2 * p
            pltpu.make_async_copy(x_hbm.at[r0 + 1], in1, si1).start()
            pltpu.make_async_copy(x_hbm.at[r0], in0, si0).wait()
            av, bv, base = row_params(r0)

            @pl.when(p > 0)
            def _():
                pltpu.make_async_copy(ob0, out_hbm.at[r0], so0).wait()

            compute_row(in0, ob0, av, bv, base)
            pltpu.make_async_copy(ob0, out_hbm.at[r0], so0).start()

            @pl.when(p < pairs - 1)
            def _():
                pltpu.make_async_copy(x_hbm.at[r0 + 2], in0, si0).start()

            pltpu.make_async_copy(x_hbm.at[r0 + 1], in1, si1).wait()
            av1, bv1, base1 = row_params(r0 + 1)

            @pl.when(p > 0)
            def _():
                pltpu.make_async_copy(ob1, out_hbm.at[r0 + 1], so1).wait()

            compute_row(in1, ob1, av1, bv1, base1)
            pltpu.make_async_copy(ob1, out_hbm.at[r0 + 1], so1).start()
            return 0

        lax.fori_loop(0, pairs, pair_body, 0)
        pltpu.make_async_copy(ob0, out_hbm.at[base_row], so0).wait()
        pltpu.make_async_copy(ob1, out_hbm.at[base_row + 1], so1).wait()

    return sc_shift


def kernel(x, shift_param):
    B, C, T = x.shape
    xr = x.reshape(B * C, T)
    sp = shift_param.reshape(C).astype(jnp.float32)
    out = _make_sc_shift(B * C, T, C)(xr, sp)
    return out.reshape(B, C, T)


# speedup vs baseline: 31.9164x; 3.1988x over previous
import functools
import jax
import jax.numpy as jnp
from jax import lax
from jax.experimental import pallas as pl
from jax.experimental.pallas import tpu as pltpu
from jax.experimental.pallas import tpu_sc as plsc

MAX_STEPS = 204.0
L = 16
HEAD = 16
TAIL = 16
U = 8


def _make_sc_shift(R, T, C):
    info = plsc.get_sparse_core_info()
    NC, NS = info.num_cores, info.num_subcores
    NW = NC * NS
    assert R % (2 * NW) == 0 and C % L == 0
    nchunks = T // L
    n_int = nchunks - HEAD - TAIL
    assert T % L == 0 and n_int % U == 0 and HEAD * L >= MAX_STEPS + 1
    rows_per = R // NW
    pairs = rows_per // 2

    mesh = plsc.VectorSubcoreMesh(core_axis_name="c", subcore_axis_name="s")

    @functools.partial(
        pl.kernel,
        mesh=mesh,
        out_type=jax.ShapeDtypeStruct((R, T), jnp.float32),
        compiler_params=pltpu.CompilerParams(needs_layout_passes=False),
        scratch_types=[
            pltpu.VMEM((C,), jnp.float32),
            pltpu.VMEM((C,), jnp.int32),
            pltpu.VMEM((C,), jnp.float32),
            pltpu.VMEM((T,), jnp.float32),
            pltpu.VMEM((T,), jnp.float32),
            pltpu.VMEM((T,), jnp.float32),
            pltpu.VMEM((T,), jnp.float32),
            pltpu.SemaphoreType.DMA,
            pltpu.SemaphoreType.DMA,
            pltpu.SemaphoreType.DMA,
            pltpu.SemaphoreType.DMA,
        ],
    )
    def sc_shift(x_hbm, shift_hbm, out_hbm, shiftv, kbuf, abuf,
                 in0, in1, ob0, ob1, si0, si1, so0, so1):
        wid = lax.axis_index("s") * NC + lax.axis_index("c")
        pltpu.sync_copy(shift_hbm, shiftv)

        for i in range(C // L):
            p = shiftv[pl.ds(i * L, L)]
            e = jnp.exp(-2.0 * jnp.abs(p))
            s = jnp.sign(p) * ((1.0 - e) / (1.0 + e)) * MAX_STEPS
            tr = s.astype(jnp.int32)
            kf = jnp.where(tr.astype(jnp.float32) > s, tr - 1, tr)
            kbuf[pl.ds(i * L, L)] = kf
            abuf[pl.ds(i * L, L)] = s - kf.astype(jnp.float32)

        iota = lax.iota(jnp.int32, L)
        base_row = wid * rows_per

        def row_params(row):
            ch = jnp.full((L,), lax.rem(row, C), jnp.int32)
            av = plsc.load_gather(abuf, [ch])
            base = plsc.load_gather(kbuf, [ch]) + iota
            return av, 1.0 - av, base

        def clip_chunk(inb, ob, av, bv, base, t):
            t = pl.multiple_of(t, L)
            idx = base + t
            i0 = jnp.minimum(jnp.maximum(idx, 0), T - 1)
            i1 = jnp.minimum(jnp.maximum(idx + 1, 0), T - 1)
            v0 = plsc.load_gather(inb, [i0])
            v1 = plsc.load_gather(inb, [i1])
            ob[pl.ds(t, L)] = bv * v0 + av * v1

        def compute_row(inb, ob, av, bv, base):
            @plsc.parallel_loop(0, HEAD, unroll=8)
            def _(ci):
                clip_chunk(inb, ob, av, bv, base, ci * L)

            @plsc.parallel_loop(HEAD, nchunks - TAIL, unroll=U)
            def _(ci):
                t = pl.multiple_of(ci * L, L)
                i0 = base + t
                v0 = plsc.load_gather(inb, [i0])
                v1 = plsc.load_gather(inb, [i0 + 1])
                ob[pl.ds(t, L)] = bv * v0 + av * v1

            @plsc.parallel_loop(nchunks - TAIL, nchunks, unroll=8)
            def _(ci):
                clip_chunk(inb, ob, av, bv, base, ci * L)

        pltpu.make_async_copy(x_hbm.at[base_row], in0, si0).start()

        def pair_body(p, _):
            r0 = base_row + 2 * p
            pltpu.make_async_copy(x_hbm.at[r0 + 1], in1, si1).start()
            pltpu.make_async_copy(x_hbm.at[r0], in0, si0).wait()
            av, bv, base = row_params(r0)

            @pl.when(p > 0)
            def _():
                pltpu.make_async_copy(ob0, out_hbm.at[r0], so0).wait()

            compute_row(in0, ob0, av, bv, base)
            pltpu.make_async_copy(ob0, out_hbm.at[r0], so0).start()

            @pl.when(p < pairs - 1)
            def _():
                pltpu.make_async_copy(x_hbm.at[r0 + 2], in0, si0).start()

            pltpu.make_async_copy(x_hbm.at[r0 + 1], in1, si1).wait()
            av1, bv1, base1 = row_params(r0 + 1)

            @pl.when(p > 0)
            def _():
                pltpu.make_async_copy(ob1, out_hbm.at[r0 + 1], so1).wait()

            compute_row(in1, ob1, av1, bv1, base1)
            pltpu.make_async_copy(ob1, out_hbm.at[r0 + 1], so1).start()
            return 0

        lax.fori_loop(0, pairs, pair_body, 0)
        pltpu.make_async_copy(ob0, out_hbm.at[base_row], so0).wait()
        pltpu.make_async_copy(ob1, out_hbm.at[base_row + 1], so1).wait()

    return sc_shift


def kernel(x, shift_param):
    B, C, T = x.shape
    xr = x.reshape(B * C, T)
    sp = shift_param.reshape(C).astype(jnp.float32)
    out = _make_sc_shift(B * C, T, C)(xr, sp)
    return out.reshape(B, C, T)
